# baseline (speedup 1.0000x reference)
import jax
import jax.numpy as jnp
from jax import lax
from jax.experimental import pallas as pl
from jax.experimental.pallas import tpu as pltpu
from jax.experimental.pallas import tpu_sc as plsc

_D = 64
_B = 100000
_NC = 2
_NS = 16
_NW = _NC * _NS
_DT = _D // 8
_NT = 782
_NF = 28
_SPAN = _NF * 128
_NB = 4


def _gather_body(idx_hbm, tab_hbm, out_hbm, tab_sh, idx_v, rb0, rb1, buf,
                 gsem, wsem):
    c = lax.axis_index("c")
    s = lax.axis_index("s")
    wid = s * _NC + c

    @pl.when(s == 0)
    def _stage_table():
        pltpu.sync_copy(tab_hbm, tab_sh)

    tb = (wid * (_NT - _NF)) // (_NW - 1)
    base = pl.multiple_of(tb * 128, 128)
    pltpu.sync_copy(idx_hbm.at[pl.ds(base, _SPAN)], idx_v)
    plsc.subcore_barrier()

    rbs = [rb0, rb1]

    def _gather(f):
        return pltpu.make_async_copy(
            tab_sh.at[idx_v.at[pl.ds(f * 128, 128)]], rbs[f % 2],
            gsem.at[f % 2])

    def _writes(f):
        slot = f % _NB
        res = []
        for dt in range(_DT):
            src = buf.at[slot, dt, :, pl.ds(0, 128)]
            dst = out_hbm.at[pl.ds((dt * _NT + tb + f) * 8, 8)]
            res.append(pltpu.make_async_copy(src, dst, wsem.at[slot]))
        return res

    lane = lax.iota(jnp.int32, 16)
    consts = []
    for k in range(4):
        consts.append((2 * k + lane // 8))
    ds_vec = lane % 8

    _gather(0).start()
    for f in range(_NF):
        if f + 1 < _NF:
            _gather(f + 1).start()
        _gather(f).wait()
        if f >= _NB:
            for cd in _writes(f - _NB):
                cd.wait()
        rb = rbs[f % 2]
        slot = f % _NB
        slot_vec = jnp.full((16,), slot, jnp.int32)

        def node_body(i, carry):
            for u in range(4):
                n = i * 4 + u
                col = jnp.full((16,), 0, jnp.int32) + n
                for k in range(4):
                    v = rb[n, pl.ds(k * 16, 16)]
                    plsc.store_scatter(buf, [slot_vec, consts[k], ds_vec, col], v)
            return carry

        lax.fori_loop(0, 32, node_body, 0)
        for cd in _writes(f):
            cd.start()

    for f in range(_NF - _NB, _NF):
        for cd in _writes(f):
            cd.wait()


@jax.jit
def _embed_lookup(idx, table):
    f = pl.kernel(
        _gather_body,
        out_type=jax.ShapeDtypeStruct((_DT * _NT * 8, 128), jnp.float32),
        mesh=plsc.VectorSubcoreMesh(core_axis_name="c", subcore_axis_name="s"),
        scratch_types=[
            pltpu.VMEM_SHARED((100, _D), jnp.float32),
            pltpu.VMEM((_SPAN,), jnp.int32),
            pltpu.VMEM((128, _D), jnp.float32),
            pltpu.VMEM((128, _D), jnp.float32),
            pltpu.VMEM((_NB, _DT, 8, 129), jnp.float32),
            pltpu.SemaphoreType.DMA((2,)),
            pltpu.SemaphoreType.DMA((_NB,)),
        ],
        compiler_params=pltpu.CompilerParams(
            use_tc_tiling_on_sc=False, needs_layout_passes=False,
            disable_bounds_checks=True, disable_semaphore_checks=True),
    )
    out = f(idx, table)
    out = out.reshape(_DT, _NT, 8, 128)
    return out.transpose(1, 3, 0, 2).reshape(_NT * 128, _D)[:_B]


def kernel(node_type, embeddings):
    idx = node_type.reshape(-1).astype(jnp.int32)
    idx = jnp.pad(idx, (0, _NT * 128 - _B))
    return _embed_lookup(idx, embeddings.astype(jnp.float32))


# speedup vs baseline: 1.4438x; 1.4438x over previous
import jax
import jax.numpy as jnp
from jax import lax
from jax.experimental import pallas as pl
from jax.experimental.pallas import tpu as pltpu
from jax.experimental.pallas import tpu_sc as plsc

_D = 64
_B = 100000
_NC = 2
_NS = 16
_NW = _NC * _NS
_DT = _D // 8
_NT = 782
_NF = 28
_SPAN = _NF * 128
_NB = 4


def _gather_body(idx_hbm, tab_hbm, out_hbm, tab_sh, idx_v, rb0, rb1, buf,
                 gsem, wsem):
    c = lax.axis_index("c")
    s = lax.axis_index("s")
    wid = s * _NC + c

    @pl.when(s == 0)
    def _stage_table():
        pltpu.sync_copy(tab_hbm, tab_sh)

    tb = (wid * (_NT - _NF)) // (_NW - 1)
    base = pl.multiple_of(tb * 128, 128)
    pltpu.sync_copy(idx_hbm.at[pl.ds(base, _SPAN)], idx_v)
    plsc.subcore_barrier()

    rbs = [rb0, rb1]

    def _gather(f):
        return pltpu.make_async_copy(
            tab_sh.at[idx_v.at[pl.ds(f * 128, 128)]], rbs[f % 2],
            gsem.at[f % 2])

    def _writes(f):
        slot = f % _NB
        res = []
        for dt in range(_DT):
            src = buf.at[slot, dt, :, pl.ds(0, 128)]
            dst = out_hbm.at[pl.ds((dt * _NT + tb + f) * 8, 8)]
            res.append(pltpu.make_async_copy(src, dst, wsem.at[slot]))
        return res

    lane = lax.iota(jnp.int32, 16)
    consts = []
    for k in range(4):
        consts.append((2 * k + lane // 8))
    ds_vec = lane % 8

    _gather(0).start()
    for f in range(_NF):
        if f + 1 < _NF:
            _gather(f + 1).start()
        _gather(f).wait()
        if f >= _NB:
            for cd in _writes(f - _NB):
                cd.wait()
        rb = rbs[f % 2]
        slot = f % _NB
        slot_vec = jnp.full((16,), slot, jnp.int32)

        @plsc.parallel_loop(0, 128, 1, unroll=4)
        def node_body(n):
            col = jnp.full((16,), 0, jnp.int32) + n
            vs = [rb[n, pl.ds(k * 16, 16)] for k in range(4)]
            for k in range(4):
                plsc.store_scatter(buf, [slot_vec, consts[k], ds_vec, col], vs[k])
        for cd in _writes(f):
            cd.start()

    for f in range(_NF - _NB, _NF):
        for cd in _writes(f):
            cd.wait()


@jax.jit
def _embed_lookup(idx, table):
    f = pl.kernel(
        _gather_body,
        out_type=jax.ShapeDtypeStruct((_DT * _NT * 8, 128), jnp.float32),
        mesh=plsc.VectorSubcoreMesh(core_axis_name="c", subcore_axis_name="s"),
        scratch_types=[
            pltpu.VMEM_SHARED((100, _D), jnp.float32),
            pltpu.VMEM((_SPAN,), jnp.int32),
            pltpu.VMEM((128, _D), jnp.float32),
            pltpu.VMEM((128, _D), jnp.float32),
            pltpu.VMEM((_NB, _DT, 8, 129), jnp.float32),
            pltpu.SemaphoreType.DMA((2,)),
            pltpu.SemaphoreType.DMA((_NB,)),
        ],
        compiler_params=pltpu.CompilerParams(
            use_tc_tiling_on_sc=False, needs_layout_passes=False,
            disable_bounds_checks=True, disable_semaphore_checks=True),
    )
    out = f(idx, table)
    out = out.reshape(_DT, _NT, 8, 128)
    return out.transpose(1, 3, 0, 2).reshape(_NT * 128, _D)[:_B]


def kernel(node_type, embeddings):
    idx = node_type.reshape(-1).astype(jnp.int32)
    idx = jnp.pad(idx, (0, _NT * 128 - _B))
    return _embed_lookup(idx, embeddings.astype(jnp.float32))
